# baseline (speedup 1.0000x reference)
import functools

import jax
import jax.numpy as jnp
from jax import lax
from jax.experimental import pallas as pl
from jax.experimental.pallas import tpu as pltpu
from jax.experimental.pallas import tpu_sc as plsc

N = 50000
M = 16
D_FEAT = 128
D_MSG = 64
D_OUT = D_FEAT + D_MSG
C = 16
NCHUNK = N // C
NW = 32
KMAX = -(-NCHUNK // NW)
L = 16


def _lane_sum(v):
    for sh in (1, 2, 4, 8):
        perm = jnp.bitwise_xor(lax.iota(jnp.int32, L), sh)
        v = v + jnp.take(v, perm)
    return v


def _in_copies(t, feat_hbm, nts_hbm, mts_hbm, msg_hbm, bufs, sems):
    msg_buf, feat_buf, nts_buf, mts_buf, _ = bufs
    base = t * C
    h = C // 2
    return (
        pltpu.make_async_copy(msg_hbm.at[pl.ds(base, h)],
                              msg_buf.at[pl.ds(0, h)], sems.at[0]),
        pltpu.make_async_copy(msg_hbm.at[pl.ds(base + h, h)],
                              msg_buf.at[pl.ds(h, h)], sems.at[4]),
        pltpu.make_async_copy(feat_hbm.at[pl.ds(base, C)], feat_buf, sems.at[1]),
        pltpu.make_async_copy(nts_hbm.at[pl.ds(base, C)], nts_buf, sems.at[2]),
        pltpu.make_async_copy(mts_hbm.at[pl.ds(base, C)], mts_buf, sems.at[3]),
    )


def _issue(t, feat_hbm, nts_hbm, mts_hbm, msg_hbm, bufs, sems):
    @pl.when(t < NCHUNK)
    def _():
        for cp in _in_copies(t, feat_hbm, nts_hbm, mts_hbm, msg_hbm, bufs, sems):
            cp.start()


def _compute_chunk(bufs):
    msg_buf, feat_buf, nts_buf, mts_buf, out_buf = bufs
    ones = jnp.ones((L,), jnp.float32)
    nts_chunk = nts_buf[...]
    for i in range(C):
        mtsv = mts_buf[i, :]
        w = jnp.exp(-jnp.abs(mtsv - nts_chunk[i]))
        rden = ones / (_lane_sum(w) + 1e-8)
        wb = [w]
        iota = lax.iota(jnp.int32, L)
        for bit in (8, 4, 2, 1):
            mask0 = (jnp.bitwise_and(iota, bit) == 0)
            nxt = []
            for u in wb:
                tu = jnp.take(u, jnp.bitwise_xor(iota, bit))
                nxt.append(jnp.where(mask0, u, tu))
                nxt.append(jnp.where(mask0, tu, u))
            wb = nxt
        acc0 = jnp.zeros((L,), jnp.float32)
        acc1 = jnp.zeros((L,), jnp.float32)
        acc2 = jnp.zeros((L,), jnp.float32)
        acc3 = jnp.zeros((L,), jnp.float32)
        for m in range(M):
            wm = wb[m]
            s = m * D_MSG
            acc0 = acc0 + msg_buf[i, pl.ds(s + 0 * L, L)] * wm
            acc1 = acc1 + msg_buf[i, pl.ds(s + 1 * L, L)] * wm
            acc2 = acc2 + msg_buf[i, pl.ds(s + 2 * L, L)] * wm
            acc3 = acc3 + msg_buf[i, pl.ds(s + 3 * L, L)] * wm
        out_buf[i, pl.ds(D_FEAT + 0 * L, L)] = acc0 * rden
        out_buf[i, pl.ds(D_FEAT + 1 * L, L)] = acc1 * rden
        out_buf[i, pl.ds(D_FEAT + 2 * L, L)] = acc2 * rden
        out_buf[i, pl.ds(D_FEAT + 3 * L, L)] = acc3 * rden
        for c8 in range(D_FEAT // L):
            out_buf[i, pl.ds(c8 * L, L)] = feat_buf[i, pl.ds(c8 * L, L)]


def _process(t, kk, feat_hbm, nts_hbm, mts_hbm, msg_hbm, out_hbm,
             bufs, sems, out_sem):
    out_buf = bufs[4]

    @pl.when(t < NCHUNK)
    def _():
        for cp in _in_copies(t, feat_hbm, nts_hbm, mts_hbm, msg_hbm, bufs, sems):
            cp.wait()

        @pl.when(kk > 0)
        def _():
            pltpu.make_async_copy(
                out_buf, out_hbm.at[pl.ds((t - 2 * NW) * C, C)], out_sem).wait()

        _compute_chunk(bufs)
        pltpu.make_async_copy(out_buf, out_hbm.at[pl.ds(t * C, C)], out_sem).start()


def _sc_body(feat_hbm, nts_hbm, mts_hbm, msg_hbm, out_hbm,
             msg_a, feat_a, nts_a, mts_a, out_a,
             msg_b, feat_b, nts_b, mts_b, out_b,
             sems_a, sems_b, out_sem_a, out_sem_b):
    cid = lax.axis_index("c")
    sid = lax.axis_index("s")
    wid = sid * 2 + cid
    bufs_a = (msg_a, feat_a, nts_a, mts_a, out_a)
    bufs_b = (msg_b, feat_b, nts_b, mts_b, out_b)

    _issue(wid, feat_hbm, nts_hbm, mts_hbm, msg_hbm, bufs_a, sems_a)

    def round_body(kk, _):
        t_a = wid + NW * (2 * kk)
        t_b = wid + NW * (2 * kk + 1)
        t_a2 = wid + NW * (2 * kk + 2)
        _issue(t_b, feat_hbm, nts_hbm, mts_hbm, msg_hbm, bufs_b, sems_b)
        _process(t_a, kk, feat_hbm, nts_hbm, mts_hbm, msg_hbm, out_hbm,
                 bufs_a, sems_a, out_sem_a)
        _issue(t_a2, feat_hbm, nts_hbm, mts_hbm, msg_hbm, bufs_a, sems_a)
        _process(t_b, kk, feat_hbm, nts_hbm, mts_hbm, msg_hbm, out_hbm,
                 bufs_b, sems_b, out_sem_b)
        return 0

    lax.fori_loop(0, KMAX // 2, round_body, 0)

    n_mine = (NCHUNK - 1 - wid) // NW + 1
    t_last = wid + NW * (n_mine - 1)
    t_prev = wid + NW * (n_mine - 2)

    @pl.when(jnp.logical_and(n_mine >= 2, lax.rem(n_mine - 2, 2) == 0))
    def _():
        pltpu.make_async_copy(out_a, out_hbm.at[pl.ds(t_prev * C, C)], out_sem_a).wait()

    @pl.when(jnp.logical_and(n_mine >= 2, lax.rem(n_mine - 2, 2) == 1))
    def _():
        pltpu.make_async_copy(out_b, out_hbm.at[pl.ds(t_prev * C, C)], out_sem_b).wait()

    @pl.when(jnp.logical_and(n_mine >= 1, lax.rem(n_mine - 1, 2) == 0))
    def _():
        pltpu.make_async_copy(out_a, out_hbm.at[pl.ds(t_last * C, C)], out_sem_a).wait()

    @pl.when(jnp.logical_and(n_mine >= 1, lax.rem(n_mine - 1, 2) == 1))
    def _():
        pltpu.make_async_copy(out_b, out_hbm.at[pl.ds(t_last * C, C)], out_sem_b).wait()


def kernel(target_node_features, node_timestamps, grouped_messages, grouped_message_timestamps):
    msgs2d = grouped_messages.reshape(N, M * D_MSG)
    mesh = plsc.VectorSubcoreMesh(core_axis_name="c", subcore_axis_name="s")
    buf_types = [
        pltpu.VMEM((C, M * D_MSG), jnp.float32),
        pltpu.VMEM((C, D_FEAT), jnp.float32),
        pltpu.VMEM((C,), jnp.float32),
        pltpu.VMEM((C, M), jnp.float32),
        pltpu.VMEM((C, D_OUT), jnp.float32),
    ]
    f = functools.partial(
        pl.kernel,
        mesh=mesh,
        out_type=jax.ShapeDtypeStruct((N, D_OUT), jnp.float32),
        scratch_types=buf_types + buf_types + [
            pltpu.SemaphoreType.DMA((5,)),
            pltpu.SemaphoreType.DMA((5,)),
            pltpu.SemaphoreType.DMA,
            pltpu.SemaphoreType.DMA,
        ],
    )(_sc_body)
    return f(target_node_features, node_timestamps, grouped_message_timestamps, msgs2d)


# speedup vs baseline: 1.0067x; 1.0067x over previous
import functools

import jax
import jax.numpy as jnp
from jax import lax
from jax.experimental import pallas as pl
from jax.experimental.pallas import tpu as pltpu
from jax.experimental.pallas import tpu_sc as plsc

N = 50000
M = 16
D_FEAT = 128
D_MSG = 64
D_OUT = D_FEAT + D_MSG
C = 16
NCHUNK = N // C
NW = 32
KMAX = -(-NCHUNK // NW)
L = 16


def _lane_sum(v):
    for sh in (1, 2, 4, 8):
        perm = jnp.bitwise_xor(lax.iota(jnp.int32, L), sh)
        v = v + jnp.take(v, perm)
    return v


def _in_copies(t, feat_hbm, nts_hbm, mts_hbm, msg_hbm, bufs, sems):
    msg_buf, feat_buf, nts_buf, mts_buf, _ = bufs
    base = t * C
    return (
        pltpu.make_async_copy(msg_hbm.at[pl.ds(base, C)], msg_buf, sems.at[0]),
        pltpu.make_async_copy(feat_hbm.at[pl.ds(base, C)], feat_buf, sems.at[1]),
        pltpu.make_async_copy(nts_hbm.at[pl.ds(base, C)], nts_buf, sems.at[2]),
        pltpu.make_async_copy(mts_hbm.at[pl.ds(base, C)], mts_buf, sems.at[3]),
    )


def _issue(t, feat_hbm, nts_hbm, mts_hbm, msg_hbm, bufs, sems):
    @pl.when(t < NCHUNK)
    def _():
        for cp in _in_copies(t, feat_hbm, nts_hbm, mts_hbm, msg_hbm, bufs, sems):
            cp.start()


def _compute_chunk(bufs):
    msg_buf, feat_buf, nts_buf, mts_buf, out_buf = bufs
    ones = jnp.ones((L,), jnp.float32)
    nts_chunk = nts_buf[...]
    for i in range(C):
        mtsv = mts_buf[i, :]
        w = jnp.exp(-jnp.abs(mtsv - nts_chunk[i]))
        rden = ones / (_lane_sum(w) + 1e-8)
        wb = [w]
        iota = lax.iota(jnp.int32, L)
        for bit in (8, 4, 2, 1):
            mask0 = (jnp.bitwise_and(iota, bit) == 0)
            nxt = []
            for u in wb:
                tu = jnp.take(u, jnp.bitwise_xor(iota, bit))
                nxt.append(jnp.where(mask0, u, tu))
                nxt.append(jnp.where(mask0, tu, u))
            wb = nxt
        acc0 = jnp.zeros((L,), jnp.float32)
        acc1 = jnp.zeros((L,), jnp.float32)
        acc2 = jnp.zeros((L,), jnp.float32)
        acc3 = jnp.zeros((L,), jnp.float32)
        for m in range(M):
            wm = wb[m]
            s = m * D_MSG
            acc0 = acc0 + msg_buf[i, pl.ds(s + 0 * L, L)] * wm
            acc1 = acc1 + msg_buf[i, pl.ds(s + 1 * L, L)] * wm
            acc2 = acc2 + msg_buf[i, pl.ds(s + 2 * L, L)] * wm
            acc3 = acc3 + msg_buf[i, pl.ds(s + 3 * L, L)] * wm
        out_buf[i, pl.ds(D_FEAT + 0 * L, L)] = acc0 * rden
        out_buf[i, pl.ds(D_FEAT + 1 * L, L)] = acc1 * rden
        out_buf[i, pl.ds(D_FEAT + 2 * L, L)] = acc2 * rden
        out_buf[i, pl.ds(D_FEAT + 3 * L, L)] = acc3 * rden
        for c8 in range(D_FEAT // L):
            out_buf[i, pl.ds(c8 * L, L)] = feat_buf[i, pl.ds(c8 * L, L)]


def _process(t, kk, feat_hbm, nts_hbm, mts_hbm, msg_hbm, out_hbm,
             bufs, sems, out_sem):
    out_buf = bufs[4]

    @pl.when(t < NCHUNK)
    def _():
        for cp in _in_copies(t, feat_hbm, nts_hbm, mts_hbm, msg_hbm, bufs, sems):
            cp.wait()

        @pl.when(kk > 0)
        def _():
            pltpu.make_async_copy(
                out_buf, out_hbm.at[pl.ds((t - 2 * NW) * C, C)], out_sem).wait()

        _compute_chunk(bufs)
        pltpu.make_async_copy(out_buf, out_hbm.at[pl.ds(t * C, C)], out_sem).start()


def _sc_body(feat_hbm, nts_hbm, mts_hbm, msg_hbm, out_hbm,
             msg_a, feat_a, nts_a, mts_a, out_a,
             msg_b, feat_b, nts_b, mts_b, out_b,
             sems_a, sems_b, out_sem_a, out_sem_b):
    cid = lax.axis_index("c")
    sid = lax.axis_index("s")
    wid = sid * 2 + cid
    bufs_a = (msg_a, feat_a, nts_a, mts_a, out_a)
    bufs_b = (msg_b, feat_b, nts_b, mts_b, out_b)

    _issue(wid, feat_hbm, nts_hbm, mts_hbm, msg_hbm, bufs_a, sems_a)

    def round_body(kk, _):
        t_a = wid + NW * (2 * kk)
        t_b = wid + NW * (2 * kk + 1)
        t_a2 = wid + NW * (2 * kk + 2)
        _issue(t_b, feat_hbm, nts_hbm, mts_hbm, msg_hbm, bufs_b, sems_b)
        _process(t_a, kk, feat_hbm, nts_hbm, mts_hbm, msg_hbm, out_hbm,
                 bufs_a, sems_a, out_sem_a)
        _issue(t_a2, feat_hbm, nts_hbm, mts_hbm, msg_hbm, bufs_a, sems_a)
        _process(t_b, kk, feat_hbm, nts_hbm, mts_hbm, msg_hbm, out_hbm,
                 bufs_b, sems_b, out_sem_b)
        return 0

    lax.fori_loop(0, KMAX // 2, round_body, 0)

    n_mine = (NCHUNK - 1 - wid) // NW + 1
    t_last = wid + NW * (n_mine - 1)
    t_prev = wid + NW * (n_mine - 2)

    @pl.when(jnp.logical_and(n_mine >= 2, lax.rem(n_mine - 2, 2) == 0))
    def _():
        pltpu.make_async_copy(out_a, out_hbm.at[pl.ds(t_prev * C, C)], out_sem_a).wait()

    @pl.when(jnp.logical_and(n_mine >= 2, lax.rem(n_mine - 2, 2) == 1))
    def _():
        pltpu.make_async_copy(out_b, out_hbm.at[pl.ds(t_prev * C, C)], out_sem_b).wait()

    @pl.when(jnp.logical_and(n_mine >= 1, lax.rem(n_mine - 1, 2) == 0))
    def _():
        pltpu.make_async_copy(out_a, out_hbm.at[pl.ds(t_last * C, C)], out_sem_a).wait()

    @pl.when(jnp.logical_and(n_mine >= 1, lax.rem(n_mine - 1, 2) == 1))
    def _():
        pltpu.make_async_copy(out_b, out_hbm.at[pl.ds(t_last * C, C)], out_sem_b).wait()


def kernel(target_node_features, node_timestamps, grouped_messages, grouped_message_timestamps):
    msgs2d = grouped_messages.reshape(N, M * D_MSG)
    mesh = plsc.VectorSubcoreMesh(core_axis_name="c", subcore_axis_name="s")
    buf_types = [
        pltpu.VMEM((C, M * D_MSG), jnp.float32),
        pltpu.VMEM((C, D_FEAT), jnp.float32),
        pltpu.VMEM((C,), jnp.float32),
        pltpu.VMEM((C, M), jnp.float32),
        pltpu.VMEM((C, D_OUT), jnp.float32),
    ]
    f = functools.partial(
        pl.kernel,
        mesh=mesh,
        out_type=jax.ShapeDtypeStruct((N, D_OUT), jnp.float32),
        scratch_types=buf_types + buf_types + [
            pltpu.SemaphoreType.DMA((4,)),
            pltpu.SemaphoreType.DMA((4,)),
            pltpu.SemaphoreType.DMA,
            pltpu.SemaphoreType.DMA,
        ],
    )(_sc_body)
    return f(target_node_features, node_timestamps, grouped_message_timestamps, msgs2d)
